# baseline (speedup 1.0000x reference)
import jax
import jax.numpy as jnp
from jax.experimental import pallas as pl

N = 2048
D = 384
CD = 2 * 49 * 3 * 3
R = 256
NB = N // R
F32 = jnp.float32
PREC = jax.lax.Precision.HIGHEST


def _dot(a, b):
    return jax.lax.dot_general(a, b, (((1,), (0,)), ((), ())),
                               precision=PREC, preferred_element_type=F32)


def _ln(x, w, b, eps=1e-3):
    m = jnp.mean(x, axis=-1, keepdims=True)
    v = jnp.mean((x - m) ** 2, axis=-1, keepdims=True)
    return (x - m) / jnp.sqrt(v + eps) * w + b


def _relu(x):
    return jnp.maximum(x, 0.0)


def _blk(i):
    return (i, 0)


def _full(i):
    return (0, 0)


def _ab_body(corr_ref, net_ref, inp_ref, kkf_ref, jjf_ref, kkb_ref, jjb_ref,
             w0_ref, b0_ref, w1_ref, b1_ref, lnw_ref, lnb_ref, w2_ref, b2_ref,
             nw_ref, nb_ref, net1_ref, ix_ref, jx_ref):
    c = _relu(_dot(corr_ref[...], w0_ref[...]) + b0_ref[...])
    c = _dot(c, w1_ref[...]) + b1_ref[...]
    c = _relu(_ln(c, lnw_ref[...], lnb_ref[...]))
    c = _dot(c, w2_ref[...]) + b2_ref[...]
    x = net_ref[...] + inp_ref[...] + c
    net1_ref[...] = _ln(x, nw_ref[...], nb_ref[...])
    kkf = kkf_ref[0, :]
    jjf = jjf_ref[0, :]
    kkb = kkb_ref[0, 0, :]
    jjb = jjb_ref[0, 0, :]
    mask = kkf[None, :] == kkb[:, None]
    colid = jax.lax.broadcasted_iota(jnp.int32, (R, N), 1)
    jjfc = jnp.broadcast_to(jjf[None, :], (R, N))
    jjbc = jjb[:, None]
    prev_vals = jnp.where(mask & (jjfc < jjbc), jjfc, 0)
    pm = jnp.max(prev_vals, axis=1, keepdims=True)
    ix = jnp.min(jnp.where(prev_vals == pm, colid, N), axis=1)
    next_vals = jnp.where(mask & (jjfc > jjbc), jjfc, N)
    nm = jnp.min(next_vals, axis=1, keepdims=True)
    jx = jnp.min(jnp.where(next_vals == nm, colid, N), axis=1)
    ix_ref[0, 0, :] = jnp.clip(ix, 0, N - 1)
    jx_ref[0, 0, :] = jnp.clip(jx, 0, N - 1)


def _gather(idx, netf):
    colid = jax.lax.broadcasted_iota(jnp.int32, (R, N), 1)
    g = (idx[:, None] == colid).astype(F32)
    return _dot(g, netf)


def _segsum(q, zf):
    b0 = pl.program_id(0) * R
    rowid = b0 + jax.lax.broadcasted_iota(jnp.int32, (R, N), 0)
    m = (q[None, :] == rowid).astype(F32)
    zsafe = jnp.where(jnp.isnan(zf), 0.0, jnp.clip(zf, -3e38, 3e38))
    return _dot(m, zsafe)


def _c1_body(netf_ref, netb_ref, ixb_ref, l1_ref, l1b_ref, l2_ref, l2b_ref,
             out_ref):
    g = _gather(ixb_ref[0, 0, :], netf_ref[...])
    h = _relu(_dot(g, l1_ref[...]) + l1b_ref[...])
    out_ref[...] = netb_ref[...] + _dot(h, l2_ref[...]) + l2b_ref[...]


def _c2d1_body(netf_ref, netb_ref, jxb_ref, l1_ref, l1b_ref, l2_ref, l2b_ref,
               wg_ref, bg_ref, wf_ref, bf_ref, net3_ref, e_ref, f_ref):
    g = _gather(jxb_ref[0, 0, :], netf_ref[...])
    h = _relu(_dot(g, l1_ref[...]) + l1b_ref[...])
    net3 = netb_ref[...] + _dot(h, l2_ref[...]) + l2b_ref[...]
    net3_ref[...] = net3
    e_ref[...] = jnp.exp(_dot(net3, wg_ref[...]) + bg_ref[...])
    f_ref[...] = _dot(net3, wf_ref[...]) + bf_ref[...]


def _d23_kk_body(kkf_ref, ef_ref, eb_ref, fb_ref, z_ref):
    q = jnp.clip(kkf_ref[0, :], 0, N - 1)
    denom = _segsum(q, ef_ref[...])
    z_ref[...] = fb_ref[...] * (eb_ref[...] / jnp.maximum(denom, 1e-6))


def _d4kk_d1ij_body(kkf_ref, iif_ref, jjf_ref, zf_ref, net3b_ref,
                    wh_ref, bh_ref, wg2_ref, bg2_ref, wf2_ref, bf2_ref,
                    net4_ref, e2_ref, f2_ref):
    q = jnp.clip(kkf_ref[0, :], 0, N - 1)
    y = _segsum(q, zf_ref[...])
    net4 = net3b_ref[...] + _dot(y, wh_ref[...]) + bh_ref[...]
    net4_ref[...] = net4
    e2_ref[...] = jnp.exp(_dot(net4, wg2_ref[...]) + bg2_ref[...])
    f2_ref[...] = _dot(net4, wf2_ref[...]) + bf2_ref[...]


def _d23_ij_body(iif_ref, jjf_ref, ef_ref, eb_ref, fb_ref, z_ref):
    q = jnp.clip(iif_ref[0, :] * 12345 + jjf_ref[0, :], 0, N - 1)
    denom = _segsum(q, ef_ref[...])
    z_ref[...] = fb_ref[...] * (eb_ref[...] / jnp.maximum(denom, 1e-6))


def _d4e_body(iif_ref, jjf_ref, zf_ref, net4b_ref, wh2_ref, bh2_ref,
              ln1w_ref, ln1b_ref, g1w_ref, g1b_ref, r11_ref, r11b_ref,
              r12_ref, r12b_ref, ln2w_ref, ln2b_ref, g2w_ref, g2b_ref,
              r21_ref, r21b_ref, r22_ref, r22b_ref, hw_ref, hb_ref,
              net5_ref, head_ref):
    q = jnp.clip(iif_ref[0, :] * 12345 + jjf_ref[0, :], 0, N - 1)
    y = _segsum(q, zf_ref[...])
    x = net4b_ref[...] + _dot(y, wh2_ref[...]) + bh2_ref[...]
    x = _ln(x, ln1w_ref[...], ln1b_ref[...])
    gate = jax.nn.sigmoid(_dot(x, g1w_ref[...]) + g1b_ref[...])
    res = _dot(_relu(_dot(x, r11_ref[...]) + r11b_ref[...]), r12_ref[...]) + r12b_ref[...]
    x = x + gate * res
    x = _ln(x, ln2w_ref[...], ln2b_ref[...])
    gate = jax.nn.sigmoid(_dot(x, g2w_ref[...]) + g2b_ref[...])
    res = _dot(_relu(_dot(x, r21_ref[...]) + r21b_ref[...]), r22_ref[...]) + r22b_ref[...]
    x = x + gate * res
    net5_ref[...] = x
    h = _dot(_relu(x), hw_ref[...]) + hb_ref[...]
    cid = jax.lax.broadcasted_iota(jnp.int32, (R, 128), 1)
    head_ref[...] = jnp.where((cid >= 2) & (cid < 4), jax.nn.sigmoid(h), h)


def _spec(shape, index_map):
    return pl.BlockSpec(shape, index_map)


_ROW = pl.BlockSpec((R, D), _blk)
_NETF = pl.BlockSpec((N, D), _full)
_W = pl.BlockSpec((D, D), _full)
_B = pl.BlockSpec((1, D), _full)
_IDXF = pl.BlockSpec((1, N), _full)
_IDXB = pl.BlockSpec((1, 1, R), lambda i: (i, 0, 0))


def _shape(s, dtype=F32):
    return jax.ShapeDtypeStruct(s, dtype)


def kernel(net, inp, corr, ii, jj, kk, params):
    p = params
    net2d = jnp.transpose(net[0, :, :, 0], (1, 0))
    inp2d = jnp.transpose(inp[0, :, :, 0], (1, 0))
    corr2d = jnp.transpose(corr[0, :, :, 0], (1, 0))
    iif = ii.astype(jnp.int32).reshape(1, N)
    jjf = jj.astype(jnp.int32).reshape(1, N)
    kkf = kk.astype(jnp.int32).reshape(1, N)
    kkb = kkf.reshape(NB, 1, R)
    jjb = jjf.reshape(NB, 1, R)

    def wT(q):
        return jnp.transpose(q["w"], (1, 0))

    def b2(q):
        return q["b"].reshape(1, -1)

    def ln2(q):
        return q["w"].reshape(1, D), q["b"].reshape(1, D)

    lnw_c, lnb_c = ln2(p["corr_ln"])
    lnw_n, lnb_n = ln2(p["norm"])

    net1, ix, jx = pl.pallas_call(
        _ab_body,
        grid=(NB,),
        in_specs=[
            pl.BlockSpec((R, CD), _blk), _ROW, _ROW,
            _IDXF, _IDXF, _IDXB, _IDXB,
            pl.BlockSpec((CD, D), _full), _B, _W, _B, _B, _B, _W, _B, _B, _B,
        ],
        out_specs=[_ROW, _IDXB, _IDXB],
        out_shape=[_shape((N, D)), _shape((NB, 1, R), jnp.int32),
                   _shape((NB, 1, R), jnp.int32)],
    )(corr2d, net2d, inp2d, kkf, jjf, kkb, jjb,
      wT(p["corr0"]), b2(p["corr0"]), wT(p["corr1"]), b2(p["corr1"]),
      lnw_c, lnb_c, wT(p["corr2"]), b2(p["corr2"]), lnw_n, lnb_n)

    net2 = pl.pallas_call(
        _c1_body,
        grid=(NB,),
        in_specs=[_NETF, _ROW, _IDXB, _W, _B, _W, _B],
        out_specs=_ROW,
        out_shape=_shape((N, D)),
    )(net1, net1, ix, wT(p["c1"]["l1"]), b2(p["c1"]["l1"]),
      wT(p["c1"]["l2"]), b2(p["c1"]["l2"]))

    net3, e1, f1 = pl.pallas_call(
        _c2d1_body,
        grid=(NB,),
        in_specs=[_NETF, _ROW, _IDXB, _W, _B, _W, _B, _W, _B, _W, _B],
        out_specs=[_ROW, _ROW, _ROW],
        out_shape=[_shape((N, D))] * 3,
    )(net2, net2, jx, wT(p["c2"]["l1"]), b2(p["c2"]["l1"]),
      wT(p["c2"]["l2"]), b2(p["c2"]["l2"]),
      wT(p["agg_kk"]["g"]), b2(p["agg_kk"]["g"]),
      wT(p["agg_kk"]["f"]), b2(p["agg_kk"]["f"]))

    z1 = pl.pallas_call(
        _d23_kk_body,
        grid=(NB,),
        in_specs=[_IDXF, _NETF, _ROW, _ROW],
        out_specs=_ROW,
        out_shape=_shape((N, D)),
    )(kkf, e1, e1, f1)

    net4, e2, f2 = pl.pallas_call(
        _d4kk_d1ij_body,
        grid=(NB,),
        in_specs=[_IDXF, _IDXF, _IDXF, _NETF, _ROW, _W, _B, _W, _B, _W, _B],
        out_specs=[_ROW, _ROW, _ROW],
        out_shape=[_shape((N, D))] * 3,
    )(kkf, iif, jjf, z1, net3,
      wT(p["agg_kk"]["h"]), b2(p["agg_kk"]["h"]),
      wT(p["agg_ij"]["g"]), b2(p["agg_ij"]["g"]),
      wT(p["agg_ij"]["f"]), b2(p["agg_ij"]["f"]))

    z2 = pl.pallas_call(
        _d23_ij_body,
        grid=(NB,),
        in_specs=[_IDXF, _IDXF, _NETF, _ROW, _ROW],
        out_specs=_ROW,
        out_shape=_shape((N, D)),
    )(iif, jjf, e2, e2, f2)

    lnw1, lnb1 = ln2(p["gru_ln1"])
    lnw2, lnb2 = ln2(p["gru_ln2"])
    hw = jnp.zeros((D, 128), F32)
    hw = hw.at[:, 0:2].set(jnp.transpose(p["d"]["w"], (1, 0)))
    hw = hw.at[:, 2:4].set(jnp.transpose(p["w"]["w"], (1, 0)))
    hb = jnp.zeros((1, 128), F32)
    hb = hb.at[0, 0:2].set(p["d"]["b"])
    hb = hb.at[0, 2:4].set(p["w"]["b"])

    net5, head = pl.pallas_call(
        _d4e_body,
        grid=(NB,),
        in_specs=[_IDXF, _IDXF, _NETF, _ROW, _W, _B,
                  _B, _B, _W, _B, _W, _B, _W, _B,
                  _B, _B, _W, _B, _W, _B, _W, _B,
                  pl.BlockSpec((D, 128), _full), pl.BlockSpec((1, 128), _full)],
        out_specs=[_ROW, pl.BlockSpec((R, 128), _blk)],
        out_shape=[_shape((N, D)), _shape((N, 128))],
    )(iif, jjf, z2, net4, wT(p["agg_ij"]["h"]), b2(p["agg_ij"]["h"]),
      lnw1, lnb1,
      wT(p["gru_gr1"]["gate"]), b2(p["gru_gr1"]["gate"]),
      wT(p["gru_gr1"]["r1"]), b2(p["gru_gr1"]["r1"]),
      wT(p["gru_gr1"]["r2"]), b2(p["gru_gr1"]["r2"]),
      lnw2, lnb2,
      wT(p["gru_gr2"]["gate"]), b2(p["gru_gr2"]["gate"]),
      wT(p["gru_gr2"]["r1"]), b2(p["gru_gr2"]["r1"]),
      wT(p["gru_gr2"]["r2"]), b2(p["gru_gr2"]["r2"]),
      hw, hb)

    net_out = net5.reshape(1, N, D)
    d_out = head[:, 0:2].reshape(1, N, 2)
    w_out = head[:, 2:4].reshape(1, N, 2)
    return net_out, d_out, w_out


# speedup vs baseline: 2.2030x; 2.2030x over previous
import jax
import jax.numpy as jnp
from jax.experimental import pallas as pl

N = 2048
D = 384
CD = 2 * 49 * 3 * 3
R = 256
NB = N // R
F32 = jnp.float32
PREC = jax.lax.Precision.DEFAULT


def _dot(a, b):
    return jax.lax.dot_general(a, b, (((1,), (0,)), ((), ())),
                               precision=PREC, preferred_element_type=F32)


def _ln(x, w, b, eps=1e-3):
    m = jnp.mean(x, axis=-1, keepdims=True)
    v = jnp.mean((x - m) ** 2, axis=-1, keepdims=True)
    return (x - m) / jnp.sqrt(v + eps) * w + b


def _relu(x):
    return jnp.maximum(x, 0.0)


def _blk(i):
    return (i, 0)


def _full(i):
    return (0, 0)


def _ab_body(corr_ref, net_ref, inp_ref, kkf_ref, jjf_ref, kkb_ref, jjb_ref,
             w0_ref, b0_ref, w1_ref, b1_ref, lnw_ref, lnb_ref, w2_ref, b2_ref,
             nw_ref, nb_ref, net1_ref, ix_ref, jx_ref):
    c = _relu(_dot(corr_ref[...], w0_ref[...]) + b0_ref[...])
    c = _dot(c, w1_ref[...]) + b1_ref[...]
    c = _relu(_ln(c, lnw_ref[...], lnb_ref[...]))
    c = _dot(c, w2_ref[...]) + b2_ref[...]
    x = net_ref[...] + inp_ref[...] + c
    net1_ref[...] = _ln(x, nw_ref[...], nb_ref[...])
    kkf = kkf_ref[0, :]
    jjf = jjf_ref[0, :]
    kkb = kkb_ref[0, 0, :]
    jjb = jjb_ref[0, 0, :]
    mask = kkf[None, :] == kkb[:, None]
    colid = jax.lax.broadcasted_iota(jnp.int32, (R, N), 1)
    jjfc = jnp.broadcast_to(jjf[None, :], (R, N))
    jjbc = jjb[:, None]
    prev_vals = jnp.where(mask & (jjfc < jjbc), jjfc, 0)
    pm = jnp.max(prev_vals, axis=1, keepdims=True)
    ix = jnp.min(jnp.where(prev_vals == pm, colid, N), axis=1)
    next_vals = jnp.where(mask & (jjfc > jjbc), jjfc, N)
    nm = jnp.min(next_vals, axis=1, keepdims=True)
    jx = jnp.min(jnp.where(next_vals == nm, colid, N), axis=1)
    ix_ref[0, 0, :] = jnp.clip(ix, 0, N - 1)
    jx_ref[0, 0, :] = jnp.clip(jx, 0, N - 1)


def _gather(idx, netf):
    colid = jax.lax.broadcasted_iota(jnp.int32, (R, N), 1)
    g = (idx[:, None] == colid).astype(F32)
    return _dot(g, netf)


def _segsum(q, zf):
    b0 = pl.program_id(0) * R
    rowid = b0 + jax.lax.broadcasted_iota(jnp.int32, (R, N), 0)
    m = (q[None, :] == rowid).astype(F32)
    zsafe = jnp.where(jnp.isnan(zf), 0.0, jnp.clip(zf, -3e38, 3e38))
    return _dot(m, zsafe)


def _c1_body(netf_ref, netb_ref, ixb_ref, l1_ref, l1b_ref, l2_ref, l2b_ref,
             out_ref):
    g = _gather(ixb_ref[0, 0, :], netf_ref[...])
    h = _relu(_dot(g, l1_ref[...]) + l1b_ref[...])
    out_ref[...] = netb_ref[...] + _dot(h, l2_ref[...]) + l2b_ref[...]


def _c2d1_body(netf_ref, netb_ref, jxb_ref, l1_ref, l1b_ref, l2_ref, l2b_ref,
               wg_ref, bg_ref, wf_ref, bf_ref, net3_ref, e_ref, f_ref):
    g = _gather(jxb_ref[0, 0, :], netf_ref[...])
    h = _relu(_dot(g, l1_ref[...]) + l1b_ref[...])
    net3 = netb_ref[...] + _dot(h, l2_ref[...]) + l2b_ref[...]
    net3_ref[...] = net3
    e_ref[...] = jnp.exp(_dot(net3, wg_ref[...]) + bg_ref[...])
    f_ref[...] = _dot(net3, wf_ref[...]) + bf_ref[...]


def _d23_kk_body(kkf_ref, ef_ref, eb_ref, fb_ref, z_ref):
    q = jnp.clip(kkf_ref[0, :], 0, N - 1)
    denom = _segsum(q, ef_ref[...])
    z_ref[...] = fb_ref[...] * (eb_ref[...] / jnp.maximum(denom, 1e-6))


def _d4kk_d1ij_body(kkf_ref, iif_ref, jjf_ref, zf_ref, net3b_ref,
                    wh_ref, bh_ref, wg2_ref, bg2_ref, wf2_ref, bf2_ref,
                    net4_ref, e2_ref, f2_ref):
    q = jnp.clip(kkf_ref[0, :], 0, N - 1)
    y = _segsum(q, zf_ref[...])
    net4 = net3b_ref[...] + _dot(y, wh_ref[...]) + bh_ref[...]
    net4_ref[...] = net4
    e2_ref[...] = jnp.exp(_dot(net4, wg2_ref[...]) + bg2_ref[...])
    f2_ref[...] = _dot(net4, wf2_ref[...]) + bf2_ref[...]


def _d23_ij_body(iif_ref, jjf_ref, ef_ref, eb_ref, fb_ref, z_ref):
    q = jnp.clip(iif_ref[0, :] * 12345 + jjf_ref[0, :], 0, N - 1)
    denom = _segsum(q, ef_ref[...])
    z_ref[...] = fb_ref[...] * (eb_ref[...] / jnp.maximum(denom, 1e-6))


def _d4e_body(iif_ref, jjf_ref, zf_ref, net4b_ref, wh2_ref, bh2_ref,
              ln1w_ref, ln1b_ref, g1w_ref, g1b_ref, r11_ref, r11b_ref,
              r12_ref, r12b_ref, ln2w_ref, ln2b_ref, g2w_ref, g2b_ref,
              r21_ref, r21b_ref, r22_ref, r22b_ref, hw_ref, hb_ref,
              net5_ref, head_ref):
    q = jnp.clip(iif_ref[0, :] * 12345 + jjf_ref[0, :], 0, N - 1)
    y = _segsum(q, zf_ref[...])
    x = net4b_ref[...] + _dot(y, wh2_ref[...]) + bh2_ref[...]
    x = _ln(x, ln1w_ref[...], ln1b_ref[...])
    gate = jax.nn.sigmoid(_dot(x, g1w_ref[...]) + g1b_ref[...])
    res = _dot(_relu(_dot(x, r11_ref[...]) + r11b_ref[...]), r12_ref[...]) + r12b_ref[...]
    x = x + gate * res
    x = _ln(x, ln2w_ref[...], ln2b_ref[...])
    gate = jax.nn.sigmoid(_dot(x, g2w_ref[...]) + g2b_ref[...])
    res = _dot(_relu(_dot(x, r21_ref[...]) + r21b_ref[...]), r22_ref[...]) + r22b_ref[...]
    x = x + gate * res
    net5_ref[...] = x
    h = _dot(_relu(x), hw_ref[...]) + hb_ref[...]
    cid = jax.lax.broadcasted_iota(jnp.int32, (R, 128), 1)
    head_ref[...] = jnp.where((cid >= 2) & (cid < 4), jax.nn.sigmoid(h), h)


def _spec(shape, index_map):
    return pl.BlockSpec(shape, index_map)


_ROW = pl.BlockSpec((R, D), _blk)
_NETF = pl.BlockSpec((N, D), _full)
_W = pl.BlockSpec((D, D), _full)
_B = pl.BlockSpec((1, D), _full)
_IDXF = pl.BlockSpec((1, N), _full)
_IDXB = pl.BlockSpec((1, 1, R), lambda i: (i, 0, 0))


def _shape(s, dtype=F32):
    return jax.ShapeDtypeStruct(s, dtype)


def kernel(net, inp, corr, ii, jj, kk, params):
    p = params
    net2d = jnp.transpose(net[0, :, :, 0], (1, 0))
    inp2d = jnp.transpose(inp[0, :, :, 0], (1, 0))
    corr2d = jnp.transpose(corr[0, :, :, 0], (1, 0))
    iif = ii.astype(jnp.int32).reshape(1, N)
    jjf = jj.astype(jnp.int32).reshape(1, N)
    kkf = kk.astype(jnp.int32).reshape(1, N)
    kkb = kkf.reshape(NB, 1, R)
    jjb = jjf.reshape(NB, 1, R)

    def wT(q):
        return jnp.transpose(q["w"], (1, 0))

    def b2(q):
        return q["b"].reshape(1, -1)

    def ln2(q):
        return q["w"].reshape(1, D), q["b"].reshape(1, D)

    lnw_c, lnb_c = ln2(p["corr_ln"])
    lnw_n, lnb_n = ln2(p["norm"])

    net1, ix, jx = pl.pallas_call(
        _ab_body,
        grid=(NB,),
        in_specs=[
            pl.BlockSpec((R, CD), _blk), _ROW, _ROW,
            _IDXF, _IDXF, _IDXB, _IDXB,
            pl.BlockSpec((CD, D), _full), _B, _W, _B, _B, _B, _W, _B, _B, _B,
        ],
        out_specs=[_ROW, _IDXB, _IDXB],
        out_shape=[_shape((N, D)), _shape((NB, 1, R), jnp.int32),
                   _shape((NB, 1, R), jnp.int32)],
    )(corr2d, net2d, inp2d, kkf, jjf, kkb, jjb,
      wT(p["corr0"]), b2(p["corr0"]), wT(p["corr1"]), b2(p["corr1"]),
      lnw_c, lnb_c, wT(p["corr2"]), b2(p["corr2"]), lnw_n, lnb_n)

    net2 = pl.pallas_call(
        _c1_body,
        grid=(NB,),
        in_specs=[_NETF, _ROW, _IDXB, _W, _B, _W, _B],
        out_specs=_ROW,
        out_shape=_shape((N, D)),
    )(net1, net1, ix, wT(p["c1"]["l1"]), b2(p["c1"]["l1"]),
      wT(p["c1"]["l2"]), b2(p["c1"]["l2"]))

    net3, e1, f1 = pl.pallas_call(
        _c2d1_body,
        grid=(NB,),
        in_specs=[_NETF, _ROW, _IDXB, _W, _B, _W, _B, _W, _B, _W, _B],
        out_specs=[_ROW, _ROW, _ROW],
        out_shape=[_shape((N, D))] * 3,
    )(net2, net2, jx, wT(p["c2"]["l1"]), b2(p["c2"]["l1"]),
      wT(p["c2"]["l2"]), b2(p["c2"]["l2"]),
      wT(p["agg_kk"]["g"]), b2(p["agg_kk"]["g"]),
      wT(p["agg_kk"]["f"]), b2(p["agg_kk"]["f"]))

    z1 = pl.pallas_call(
        _d23_kk_body,
        grid=(NB,),
        in_specs=[_IDXF, _NETF, _ROW, _ROW],
        out_specs=_ROW,
        out_shape=_shape((N, D)),
    )(kkf, e1, e1, f1)

    net4, e2, f2 = pl.pallas_call(
        _d4kk_d1ij_body,
        grid=(NB,),
        in_specs=[_IDXF, _IDXF, _IDXF, _NETF, _ROW, _W, _B, _W, _B, _W, _B],
        out_specs=[_ROW, _ROW, _ROW],
        out_shape=[_shape((N, D))] * 3,
    )(kkf, iif, jjf, z1, net3,
      wT(p["agg_kk"]["h"]), b2(p["agg_kk"]["h"]),
      wT(p["agg_ij"]["g"]), b2(p["agg_ij"]["g"]),
      wT(p["agg_ij"]["f"]), b2(p["agg_ij"]["f"]))

    z2 = pl.pallas_call(
        _d23_ij_body,
        grid=(NB,),
        in_specs=[_IDXF, _IDXF, _NETF, _ROW, _ROW],
        out_specs=_ROW,
        out_shape=_shape((N, D)),
    )(iif, jjf, e2, e2, f2)

    lnw1, lnb1 = ln2(p["gru_ln1"])
    lnw2, lnb2 = ln2(p["gru_ln2"])
    hw = jnp.zeros((D, 128), F32)
    hw = hw.at[:, 0:2].set(jnp.transpose(p["d"]["w"], (1, 0)))
    hw = hw.at[:, 2:4].set(jnp.transpose(p["w"]["w"], (1, 0)))
    hb = jnp.zeros((1, 128), F32)
    hb = hb.at[0, 0:2].set(p["d"]["b"])
    hb = hb.at[0, 2:4].set(p["w"]["b"])

    net5, head = pl.pallas_call(
        _d4e_body,
        grid=(NB,),
        in_specs=[_IDXF, _IDXF, _NETF, _ROW, _W, _B,
                  _B, _B, _W, _B, _W, _B, _W, _B,
                  _B, _B, _W, _B, _W, _B, _W, _B,
                  pl.BlockSpec((D, 128), _full), pl.BlockSpec((1, 128), _full)],
        out_specs=[_ROW, pl.BlockSpec((R, 128), _blk)],
        out_shape=[_shape((N, D)), _shape((N, 128))],
    )(iif, jjf, z2, net4, wT(p["agg_ij"]["h"]), b2(p["agg_ij"]["h"]),
      lnw1, lnb1,
      wT(p["gru_gr1"]["gate"]), b2(p["gru_gr1"]["gate"]),
      wT(p["gru_gr1"]["r1"]), b2(p["gru_gr1"]["r1"]),
      wT(p["gru_gr1"]["r2"]), b2(p["gru_gr1"]["r2"]),
      lnw2, lnb2,
      wT(p["gru_gr2"]["gate"]), b2(p["gru_gr2"]["gate"]),
      wT(p["gru_gr2"]["r1"]), b2(p["gru_gr2"]["r1"]),
      wT(p["gru_gr2"]["r2"]), b2(p["gru_gr2"]["r2"]),
      hw, hb)

    net_out = net5.reshape(1, N, D)
    d_out = head[:, 0:2].reshape(1, N, 2)
    w_out = head[:, 2:4].reshape(1, N, 2)
    return net_out, d_out, w_out


# speedup vs baseline: 2.2399x; 1.0168x over previous
import jax
import jax.numpy as jnp
from jax.experimental import pallas as pl
from jax.experimental.pallas import tpu as pltpu

N = 2048
D = 384
CD = 2 * 49 * 3 * 3
R = 256
NB = N // R
F32 = jnp.float32
PREC = jax.lax.Precision.DEFAULT


def _dot(a, b):
    return jax.lax.dot_general(a, b, (((1,), (0,)), ((), ())),
                               precision=PREC, preferred_element_type=F32)


def _ln(x, w, b, eps=1e-3):
    m = jnp.mean(x, axis=-1, keepdims=True)
    v = jnp.mean((x - m) ** 2, axis=-1, keepdims=True)
    return (x - m) / jnp.sqrt(v + eps) * w[None, :] + b[None, :]


def _relu(x):
    return jnp.maximum(x, 0.0)


def _gather(idx, netf):
    colid = jax.lax.broadcasted_iota(jnp.int32, (R, N), 1)
    g = (idx[:, None] == colid).astype(F32)
    return _dot(g, netf)


def _segsum(blk, q, zf):
    rowid = blk * R + jax.lax.broadcasted_iota(jnp.int32, (R, N), 0)
    m = (q[None, :] == rowid).astype(F32)
    zsafe = jnp.where(jnp.isnan(zf), 0.0, jnp.clip(zf, -3e38, 3e38))
    return _dot(m, zsafe)


def _mega(corr_ref, netin_ref, inpin_ref, kkf_ref, jjf_ref, iif_ref,
          kkb_ref, jjb_ref, w0_ref, wstk_ref, hw_ref, bias_ref,
          net5_ref, head_ref,
          sA, sB, sC, sD, sE, ix_ref, jx_ref):
    s = pl.program_id(0)
    stage = s // NB
    blk = s % NB
    start = pl.multiple_of(blk * R, R)
    rows = pl.ds(start, R)

    def W(k):
        return wstk_ref[k]

    def bias(k):
        return bias_ref[k, :]

    @pl.when(stage == 0)
    def _s0():
        c = _relu(_dot(corr_ref[...], w0_ref[...]) + bias(0)[None, :])
        c = _dot(c, W(0)) + bias(1)[None, :]
        c = _relu(_ln(c, bias(2), bias(3)))
        c = _dot(c, W(1)) + bias(4)[None, :]
        x = netin_ref[...] + inpin_ref[...] + c
        sA[rows, :] = _ln(x, bias(5), bias(6))
        kkf = kkf_ref[0, :]
        jjf = jjf_ref[0, :]
        kkb = kkb_ref[0, 0, :]
        jjb = jjb_ref[0, 0, :]
        mask = kkf[None, :] == kkb[:, None]
        colid = jax.lax.broadcasted_iota(jnp.int32, (R, N), 1)
        jjfc = jnp.broadcast_to(jjf[None, :], (R, N))
        jjbc = jjb[:, None]
        prev_vals = jnp.where(mask & (jjfc < jjbc), jjfc, 0)
        pm = jnp.max(prev_vals, axis=1, keepdims=True)
        ixv = jnp.min(jnp.where(prev_vals == pm, colid, N), axis=1)
        next_vals = jnp.where(mask & (jjfc > jjbc), jjfc, N)
        nm = jnp.min(next_vals, axis=1, keepdims=True)
        jxv = jnp.min(jnp.where(next_vals == nm, colid, N), axis=1)
        ix_ref[blk, :] = jnp.clip(ixv, 0, N - 1)
        jx_ref[blk, :] = jnp.clip(jxv, 0, N - 1)

    @pl.when(stage == 1)
    def _s1():
        g = _gather(ix_ref[blk, :], sA[...])
        h = _relu(_dot(g, W(2)) + bias(7)[None, :])
        sB[rows, :] = sA[rows, :] + _dot(h, W(3)) + bias(8)[None, :]

    @pl.when(stage == 2)
    def _s2():
        g = _gather(jx_ref[blk, :], sB[...])
        h = _relu(_dot(g, W(4)) + bias(9)[None, :])
        net3 = sB[rows, :] + _dot(h, W(5)) + bias(10)[None, :]
        sC[rows, :] = net3
        sD[rows, :] = jnp.exp(_dot(net3, W(6)) + bias(11)[None, :])
        sE[rows, :] = _dot(net3, W(7)) + bias(12)[None, :]

    @pl.when(stage == 3)
    def _s3():
        q = jnp.clip(kkf_ref[0, :], 0, N - 1)
        denom = _segsum(blk, q, sD[...])
        sA[rows, :] = sE[rows, :] * (sD[rows, :] / jnp.maximum(denom, 1e-6))

    @pl.when(stage == 4)
    def _s4():
        q = jnp.clip(kkf_ref[0, :], 0, N - 1)
        y = _segsum(blk, q, sA[...])
        net4 = sC[rows, :] + _dot(y, W(8)) + bias(13)[None, :]
        sD[rows, :] = net4
        sB[rows, :] = jnp.exp(_dot(net4, W(9)) + bias(14)[None, :])
        sE[rows, :] = _dot(net4, W(10)) + bias(15)[None, :]

    @pl.when(stage == 5)
    def _s5():
        q2 = jnp.clip(iif_ref[0, :] * 12345 + jjf_ref[0, :], 0, N - 1)
        denom = _segsum(blk, q2, sB[...])
        sA[rows, :] = sE[rows, :] * (sB[rows, :] / jnp.maximum(denom, 1e-6))

    @pl.when(stage == 6)
    def _s6():
        q2 = jnp.clip(iif_ref[0, :] * 12345 + jjf_ref[0, :], 0, N - 1)
        y = _segsum(blk, q2, sA[...])
        x = sD[rows, :] + _dot(y, W(11)) + bias(16)[None, :]
        x = _ln(x, bias(17), bias(18))
        gate = jax.nn.sigmoid(_dot(x, W(12)) + bias(19)[None, :])
        res = _dot(_relu(_dot(x, W(13)) + bias(20)[None, :]), W(14)) + bias(21)[None, :]
        x = x + gate * res
        x = _ln(x, bias(22), bias(23))
        gate = jax.nn.sigmoid(_dot(x, W(15)) + bias(24)[None, :])
        res = _dot(_relu(_dot(x, W(16)) + bias(25)[None, :]), W(17)) + bias(26)[None, :]
        x = x + gate * res
        net5_ref[...] = x
        h = _dot(_relu(x), hw_ref[...]) + bias(27)[:128][None, :]
        cid = jax.lax.broadcasted_iota(jnp.int32, (R, 128), 1)
        head_ref[...] = jnp.where((cid >= 2) & (cid < 4), jax.nn.sigmoid(h), h)


def _shape(s, dtype=F32):
    return jax.ShapeDtypeStruct(s, dtype)


def kernel(net, inp, corr, ii, jj, kk, params):
    p = params
    net2d = jnp.transpose(net[0, :, :, 0], (1, 0))
    inp2d = jnp.transpose(inp[0, :, :, 0], (1, 0))
    corr2d = jnp.transpose(corr[0, :, :, 0], (1, 0))
    iif = ii.astype(jnp.int32).reshape(1, N)
    jjf = jj.astype(jnp.int32).reshape(1, N)
    kkf = kk.astype(jnp.int32).reshape(1, N)
    kkb = kkf.reshape(NB, 1, R)
    jjb = jjf.reshape(NB, 1, R)

    def wT(q):
        return jnp.transpose(q["w"], (1, 0))

    wstk = jnp.stack([
        wT(p["corr1"]), wT(p["corr2"]),
        wT(p["c1"]["l1"]), wT(p["c1"]["l2"]),
        wT(p["c2"]["l1"]), wT(p["c2"]["l2"]),
        wT(p["agg_kk"]["g"]), wT(p["agg_kk"]["f"]), wT(p["agg_kk"]["h"]),
        wT(p["agg_ij"]["g"]), wT(p["agg_ij"]["f"]), wT(p["agg_ij"]["h"]),
        wT(p["gru_gr1"]["gate"]), wT(p["gru_gr1"]["r1"]), wT(p["gru_gr1"]["r2"]),
        wT(p["gru_gr2"]["gate"]), wT(p["gru_gr2"]["r1"]), wT(p["gru_gr2"]["r2"]),
    ])
    hw = jnp.zeros((D, 128), F32)
    hw = hw.at[:, 0:2].set(wT(p["d"]))
    hw = hw.at[:, 2:4].set(wT(p["w"]))
    hb = jnp.zeros((D,), F32)
    hb = hb.at[0:2].set(p["d"]["b"])
    hb = hb.at[2:4].set(p["w"]["b"])
    bias = jnp.stack([
        p["corr0"]["b"], p["corr1"]["b"], p["corr_ln"]["w"], p["corr_ln"]["b"],
        p["corr2"]["b"], p["norm"]["w"], p["norm"]["b"],
        p["c1"]["l1"]["b"], p["c1"]["l2"]["b"],
        p["c2"]["l1"]["b"], p["c2"]["l2"]["b"],
        p["agg_kk"]["g"]["b"], p["agg_kk"]["f"]["b"], p["agg_kk"]["h"]["b"],
        p["agg_ij"]["g"]["b"], p["agg_ij"]["f"]["b"], p["agg_ij"]["h"]["b"],
        p["gru_ln1"]["w"], p["gru_ln1"]["b"],
        p["gru_gr1"]["gate"]["b"], p["gru_gr1"]["r1"]["b"], p["gru_gr1"]["r2"]["b"],
        p["gru_ln2"]["w"], p["gru_ln2"]["b"],
        p["gru_gr2"]["gate"]["b"], p["gru_gr2"]["r1"]["b"], p["gru_gr2"]["r2"]["b"],
        hb,
    ])

    def _in_blk(i):
        return (jnp.minimum(i, NB - 1), 0)

    def _in_blk3(i):
        return (jnp.minimum(i, NB - 1), 0, 0)

    def _out_blk(i):
        return (jnp.maximum(i - 6 * NB, 0), 0)

    def _c0(i):
        return (0, 0)

    def _c03(i):
        return (0, 0, 0)

    grid = (7 * NB,)
    net5, head = pl.pallas_call(
        _mega,
        grid=grid,
        in_specs=[
            pl.BlockSpec((R, CD), _in_blk),
            pl.BlockSpec((R, D), _in_blk),
            pl.BlockSpec((R, D), _in_blk),
            pl.BlockSpec((1, N), _c0),
            pl.BlockSpec((1, N), _c0),
            pl.BlockSpec((1, N), _c0),
            pl.BlockSpec((1, 1, R), _in_blk3),
            pl.BlockSpec((1, 1, R), _in_blk3),
            pl.BlockSpec((CD, D), _c0),
            pl.BlockSpec((18, D, D), _c03),
            pl.BlockSpec((D, 128), _c0),
            pl.BlockSpec((28, D), _c0),
        ],
        out_specs=[
            pl.BlockSpec((R, D), _out_blk),
            pl.BlockSpec((R, 128), _out_blk),
        ],
        out_shape=[_shape((N, D)), _shape((N, 128))],
        scratch_shapes=[
            pltpu.VMEM((N, D), F32), pltpu.VMEM((N, D), F32),
            pltpu.VMEM((N, D), F32), pltpu.VMEM((N, D), F32),
            pltpu.VMEM((N, D), F32),
            pltpu.VMEM((NB, R), jnp.int32), pltpu.VMEM((NB, R), jnp.int32),
        ],
    )(corr2d, net2d, inp2d, kkf, jjf, iif, kkb, jjb,
      wT(p["corr0"]), wstk, hw, bias)

    net_out = net5.reshape(1, N, D)
    d_out = head[:, 0:2].reshape(1, N, 2)
    w_out = head[:, 2:4].reshape(1, N, 2)
    return net_out, d_out, w_out
